# baseline (speedup 1.0000x reference)
import functools

import jax
import jax.numpy as jnp
import numpy as np
from jax import lax
from jax.experimental import pallas as pl
from jax.experimental.pallas import tpu as pltpu
from jax.experimental.pallas import tpu_sc as plsc

HID = 128
NH = 4
DH = HID // NH
LW = HID + 16

NC, NS = 2, 16
NW = NC * NS

BB = 400

A1, OFF_SB1, OFF_SI1, PAD1 = 12288, 10000, 11200, 11600
NB_S, NB_B, NB_I = 25, 3, 1
NB1 = NB_S + NB_B + NB_I
A2, OFF_IS2, PAD2 = 2048, 1200, 1600
A3, PAD3 = 1024, 400

_f32 = jnp.float32


def _round_up(x, m):
    return (x + m - 1) // m * m


_G = np.kron(np.eye(NH), np.ones((DH, DH))).astype(np.float32)
_P = np.kron(np.eye(NH), np.ones((DH, NH)) / DH).astype(np.float32)
_Q = np.kron(np.eye(NH), np.ones((NH, DH)) / NH).astype(np.float32)
_H = (np.tile(np.eye(DH), (NH, 1)) / NH).astype(np.float32)

_TBL_KEYS = ('wsT', 'bs', 'wmbig', 'bmrep', 'warep')
_RGA_KEYS = ('m1', 'bmerge', 'woaT', 'wobT', 'bout', 'g', 'b')


def _prep_rga(p):
    ws, bs = p['src_proj']
    wm, bm = p['msg_fc']
    wa = p['attn_fc'][0][0]
    wmerge, bmerge = p['merge_fc']
    wo, bo = p['out_fc']
    g, b = p['norm']
    eye = jnp.eye(NH, dtype=_f32)
    return dict(
        wsT=ws.T,
        bs=bs[None, :],
        wmbig=jnp.kron(eye, wm.T),
        bmrep=jnp.tile(bm, NH)[None, :],
        warep=jnp.tile(wa[:DH], NH)[None, :],
        m1=_H @ wmerge.T,
        bmerge=bmerge[None, :],
        woaT=wo[:, :HID].T,
        wobT=wo[:, HID:].T,
        bout=bo[None, :],
        g=g[None, :],
        b=b[None, :],
    )


def _stack(ws, keys):
    return tuple(jnp.stack([w[k] for w in ws]) for k in keys)


def _full(a):
    return pl.BlockSpec(a.shape, lambda i: (0,) * a.ndim)


def _wspec(a, relmap):
    return pl.BlockSpec((1,) + a.shape[1:], lambda i: (relmap(i),) + (0,) * (a.ndim - 1))


def _enc_body(x, w1T, b1, w2T, b2, o):
    h = jnp.maximum(x[...] @ w1T[0] + b1[0], 0.0)
    o[...] = h @ w2T[0] + b2[0]


def _enc_all(xfeat, params):
    def padw(p, f):
        (w1, b1), (w2, b2) = p
        w1T = jnp.pad(w1.T, ((0, 64 - w1.shape[1]), (0, 0)))
        return w1T, b1[None, :], w2.T, b2[None, :]

    stacked = [padw(params[k], 64) for k in ('stock_enc', 'bank_enc', 'industry_enc')]
    w1T, b1, w2T, b2 = (jnp.stack([s[j] for s in stacked]) for j in range(4))
    relmap = lambda i: jnp.where(i < NB_S, 0, jnp.where(i < NB_S + NB_B, 1, 2))
    return pl.pallas_call(
        _enc_body,
        grid=(NB1,),
        in_specs=[pl.BlockSpec((BB, 64), lambda i: (i, 0)),
                  _wspec(w1T, relmap), _wspec(b1, relmap),
                  _wspec(w2T, relmap), _wspec(b2, relmap)],
        out_specs=pl.BlockSpec((BB, HID), lambda i: (i, 0)),
        out_shape=jax.ShapeDtypeStruct((A1, HID), _f32),
    )(xfeat, w1T, b1, w2T, b2)


def _tbl_body(x, wsT, bs, wmbig, bmrep, warep, G, P, o):
    sf = x[...] @ wsT[0] + bs[0]
    t = jnp.tanh(sf)
    ab = (t * warep[0]) @ G[...]
    scale = jnp.exp(ab)
    wmsg = (sf @ wmbig[0] + bmrep[0]) * scale
    den16 = scale @ P[...]
    o[...] = jnp.concatenate([wmsg, den16], axis=1)


def _tbl(x, wlist, relmap, xmap, nblocks, out_rows):
    ws = _stack(wlist, _TBL_KEYS)
    return pl.pallas_call(
        _tbl_body,
        grid=(nblocks,),
        in_specs=[pl.BlockSpec((BB, HID), lambda i: (xmap(i), 0))]
        + [_wspec(a, relmap) for a in ws] + [_full(_G), _full(_P)],
        out_specs=pl.BlockSpec((BB, LW), lambda i: (i, 0)),
        out_shape=jax.ShapeDtypeStruct((out_rows, LW), _f32),
    )(x, *ws, _G, _P)


def _epi_math(acc, x, w, q):
    den_b = acc[:, HID:] @ q + 1e-12
    r = acc[:, :HID] / den_b
    merge = r @ w['m1'] + w['bmerge']
    upd = x @ w['woaT'] + merge @ w['wobT'] + w['bout']
    y = x + upd
    mu = jnp.mean(y, -1, keepdims=True)
    var = jnp.mean((y - mu) ** 2, -1, keepdims=True)
    return (y - mu) * lax.rsqrt(var + 1e-5) * w['g'] + w['b']


def _epi_body(x, a0, a1, q, m1, bmerge, woaT, wobT, bout, g, b, o):
    w = dict(m1=m1[0], bmerge=bmerge[0], woaT=woaT[0], wobT=wobT[0],
             bout=bout[0], g=g[0], b=b[0])
    o[...] = _epi_math(a0[0] + a1[0], x[...], w, q[...])


def _epi(state, acc, wlist, relmap, accmap, stmap, nblocks):
    ws = _stack(wlist, _RGA_KEYS)
    return pl.pallas_call(
        _epi_body,
        grid=(nblocks,),
        in_specs=[
            pl.BlockSpec((BB, HID), lambda i: (stmap(i), 0)),
            pl.BlockSpec((1, BB, LW), lambda i: (0, accmap(i), 0)),
            pl.BlockSpec((1, BB, LW), lambda i: (1, accmap(i), 0)),
            _full(_Q),
        ] + [_wspec(a, relmap) for a in ws],
        out_specs=pl.BlockSpec((BB, HID), lambda i: (stmap(i), 0)),
        out_shape=jax.ShapeDtypeStruct(state.shape, _f32),
        input_output_aliases={0: 0},
    )(state, acc, acc, _Q, *ws)


def _mega_body(x, ss0, ss1, sb0, sb1, si0, si1, q, *flat, nsb, nsi):
    wss = dict(zip(_RGA_KEYS, (f[...] for f in flat[0:7])))
    wsb = dict(zip(_RGA_KEYS, (f[...] for f in flat[7:14])))
    wsi = dict(zip(_RGA_KEYS, (f[...] for f in flat[14:21])))
    wpT, bp, wsrep, woT, bo, gm, bm = flat[21:28]
    o = flat[28]
    i = pl.program_id(0)
    msb = jnp.where(i < nsb, 1.0, 0.0).astype(_f32)
    msi = jnp.where(i < nsi, 1.0, 0.0).astype(_f32)
    xv = x[...]
    qv = q[...]
    p0 = xv
    p1 = _epi_math(ss0[0] + ss1[0], xv, wss, qv)
    p2 = _epi_math((sb0[0] + sb1[0]) * msb, xv, wsb, qv)
    p3 = _epi_math((si0[0] + si1[0]) * msi, xv, wsi, qv)
    paths = (p0, p1, p2, p3)
    scores = [jnp.sum(jnp.tanh(pp @ wpT[...] + bp[...]) * wsrep[...], -1, keepdims=True)
              for pp in paths]
    m = jnp.maximum(jnp.maximum(scores[0], scores[1]),
                    jnp.maximum(scores[2], scores[3]))
    es = [jnp.exp(s - m) for s in scores]
    den = es[0] + es[1] + es[2] + es[3]
    mix = sum(e * pp for e, pp in zip(es, paths)) / den
    y = mix @ woT[...] + bo[...]
    mu = jnp.mean(y, -1, keepdims=True)
    var = jnp.mean((y - mu) ** 2, -1, keepdims=True)
    o[...] = (y - mu) * lax.rsqrt(var + 1e-5) * gm[...] + bm[...]


def _mega(state, acc, wss, wsb, wsi, meta):
    sb0, si0 = OFF_SB1 // BB, OFF_SI1 // BB
    wargs = (tuple(wss[k] for k in _RGA_KEYS) + tuple(wsb[k] for k in _RGA_KEYS)
             + tuple(wsi[k] for k in _RGA_KEYS)
             + (meta['path_fc'][0].T, meta['path_fc'][1][None, :],
                meta['score_fc'][0][None, :], meta['out_fc'][0].T,
                meta['out_fc'][1][None, :], meta['norm'][0][None, :],
                meta['norm'][1][None, :]))
    return pl.pallas_call(
        functools.partial(_mega_body, nsb=NB_B, nsi=NB_I),
        grid=(NB_S,),
        in_specs=[
            pl.BlockSpec((BB, HID), lambda i: (i, 0)),
            pl.BlockSpec((1, BB, LW), lambda i: (0, i, 0)),
            pl.BlockSpec((1, BB, LW), lambda i: (1, i, 0)),
            pl.BlockSpec((1, BB, LW), lambda i: (0, sb0 + jnp.minimum(i, NB_B - 1), 0)),
            pl.BlockSpec((1, BB, LW), lambda i: (1, sb0 + jnp.minimum(i, NB_B - 1), 0)),
            pl.BlockSpec((1, BB, LW), lambda i: (0, si0 + jnp.minimum(i, NB_I - 1), 0)),
            pl.BlockSpec((1, BB, LW), lambda i: (1, si0 + jnp.minimum(i, NB_I - 1), 0)),
            _full(_Q),
        ] + [_full(a) for a in wargs],
        out_specs=pl.BlockSpec((BB, HID), lambda i: (i, 0)),
        out_shape=jax.ShapeDtypeStruct(state.shape, _f32),
        input_output_aliases={0: 0},
    )(state, acc, acc, acc, acc, acc, acc, _Q, *wargs)


def _win_for(A):
    return 64 if A > 4096 else 128


@functools.cache
def _make_scagg(A, E):
    win = _win_for(A)
    C = E // NW
    nwin = C // win
    assert C % win == 0 and nwin % 2 == 0
    niter = nwin // 2
    rz = A // NS
    assert rz % 8 == 0
    zc = next(c for c in (128, 64, 32, 16, 8) if rz % c == 0 and c <= win)
    mesh = plsc.VectorSubcoreMesh(core_axis_name="c", subcore_axis_name="s",
                                  num_cores=NC, num_subcores=NS)

    @functools.partial(
        pl.kernel,
        out_type=jax.ShapeDtypeStruct((NC, A, LW), _f32),
        mesh=mesh,
        compiler_params=pltpu.CompilerParams(use_tc_tiling_on_sc=False),
        scratch_types=[
            pltpu.VMEM((2, 2, win), jnp.int32),
            pltpu.VMEM((win, LW), _f32),
            pltpu.VMEM((win, LW), _f32),
            pltpu.VMEM_SHARED((A, LW), _f32),
            pltpu.SemaphoreType.DMA,
            pltpu.SemaphoreType.DMA,
            pltpu.SemaphoreType.DMA,
            pltpu.SemaphoreType.DMA,
        ],
    )
    def scagg(table, idxs, zrows, out, idxb, rows0, rows1, accum, g0, g1, s0, s1):
        c = lax.axis_index("c")
        s = lax.axis_index("s")
        wid = s * NC + c
        pltpu.sync_copy(zrows.at[pl.ds(0, zc)], rows0.at[pl.ds(0, zc)])
        for j in range(rz // zc):
            pltpu.sync_copy(rows0.at[pl.ds(0, zc)],
                            accum.at[pl.ds(s * rz + j * zc, zc)])
        plsc.subcore_barrier()
        sbase = wid * nwin
        pltpu.sync_copy(idxs.at[sbase], idxb.at[0])
        pltpu.async_copy(table.at[idxb.at[0, 0]], rows0, g0)

        def body(jj, carry):
            @pl.when(jj > 0)
            def _():
                pltpu.make_async_copy(table.at[pl.ds(0, win)], rows1, s1).wait()

            pltpu.sync_copy(idxs.at[sbase + 2 * jj + 1], idxb.at[1])
            pltpu.async_copy(table.at[idxb.at[1, 0]], rows1, g1)
            pltpu.make_async_copy(table.at[pl.ds(0, win)], rows0, g0).wait()
            pltpu.async_copy(rows0, accum.at[idxb.at[0, 1]], s0, add=True)

            @pl.when(jj < niter - 1)
            def _():
                pltpu.make_async_copy(table.at[pl.ds(0, win)], rows0, s0).wait()
                pltpu.sync_copy(idxs.at[sbase + 2 * jj + 2], idxb.at[0])
                pltpu.async_copy(table.at[idxb.at[0, 0]], rows0, g0)

            pltpu.make_async_copy(table.at[pl.ds(0, win)], rows1, g1).wait()
            pltpu.async_copy(rows1, accum.at[idxb.at[1, 1]], s1, add=True)
            return carry

        lax.fori_loop(0, niter, body, 0)
        pltpu.make_async_copy(table.at[pl.ds(0, win)], rows0, s0).wait()
        pltpu.make_async_copy(table.at[pl.ds(0, win)], rows1, s1).wait()
        plsc.subcore_barrier()
        pltpu.sync_copy(accum.at[pl.ds(s * rz, rz)], out.at[c, pl.ds(s * rz, rz)])

    return scagg


def _scagg(table, src, dst, A, E):
    win = _win_for(A)
    idxs = jnp.stack([src.reshape(-1, win), dst.reshape(-1, win)], axis=1)
    zrows = jnp.zeros((128, LW), _f32)
    return _make_scagg(A, E)(table, idxs, zrows)


def _cat_edges(parts, pad_base, total):
    srcs = [e[0].astype(jnp.int32) + off for e, off in parts]
    dsts = [e[1].astype(jnp.int32) + off for e, off in parts]
    ne = sum(s.shape[0] for s in srcs)
    npad = total - ne
    pad = pad_base + (jnp.arange(npad, dtype=jnp.int32) % 8)
    return jnp.concatenate(srcs + [pad]), jnp.concatenate(dsts + [pad])


def kernel(stock_feat, bank_feat, industry_feat, edge_index_ss, edge_index_sb,
           edge_index_si, edge_index_bs, edge_index_is, edge_index_ii, params):
    xfeat = jnp.concatenate([
        stock_feat,
        jnp.pad(bank_feat, ((0, 200), (0, 32))),
        jnp.pad(industry_feat, ((0, 300), (0, 48))),
        jnp.zeros((A1 - PAD1, 64), _f32),
    ])
    state = _enc_all(xfeat, params)

    e1 = _round_up(edge_index_ss.shape[1] + edge_index_sb.shape[1]
                   + edge_index_si.shape[1], NW * 2 * _win_for(A1))
    src1, dst1 = _cat_edges([(edge_index_ss, 0), (edge_index_sb, OFF_SB1),
                             (edge_index_si, OFF_SI1)], PAD1, e1)
    e2 = _round_up(edge_index_bs.shape[1] + edge_index_is.shape[1],
                   NW * 2 * _win_for(A2))
    src2, dst2 = _cat_edges([(edge_index_bs, 0), (edge_index_is, OFF_IS2)], PAD2, e2)
    e3 = _round_up(edge_index_ii.shape[1], NW * 2 * _win_for(A3))
    src3, dst3 = _cat_edges([(edge_index_ii, 0)], PAD3, e3)

    rel1 = lambda i: jnp.where(i < NB_S, 0, jnp.where(i < NB_S + NB_B, 1, 2))
    rel2 = lambda i: jnp.where(i < 3, 0, 1)
    x2map = lambda i: jnp.where(i < 3, i, 0)

    layers = params['layers']
    for li, lp in enumerate(layers):
        wss, wsb, wsi = _prep_rga(lp['SS']), _prep_rga(lp['SB']), _prep_rga(lp['SI'])

        t1 = _tbl(state, [wss, wsb, wsi], rel1, lambda i: i, NB1, A1)
        acc1 = _scagg(t1, src1, dst1, A1, e1)
        state = _mega(state, acc1, wss, wsb, wsi, params['meta'])

        if li == len(layers) - 1:
            break
        wbs, wis, wii = _prep_rga(lp['BS']), _prep_rga(lp['IS']), _prep_rga(lp['II'])

        t2 = _tbl(state, [wbs, wis], rel2, x2map, 4, A2)
        acc2 = _scagg(t2, src2, dst2, A2, e2)
        state = _epi(state, acc2, [wbs, wis], rel2, lambda i: i,
                     lambda i: NB_S + i, 4)

        t3 = _tbl(state, [wii], lambda i: 0, lambda i: NB_S + NB_B, 1, A3)
        acc3 = _scagg(t3, src3, dst3, A3, e3)
        state = _epi(state, acc3, [wii], lambda i: 0, lambda i: 0,
                     lambda i: NB_S + NB_B, 1)

    return state[:OFF_SB1]


# speedup vs baseline: 1.0825x; 1.0825x over previous
import functools

import jax
import jax.numpy as jnp
import numpy as np
from jax import lax
from jax.experimental import pallas as pl
from jax.experimental.pallas import tpu as pltpu
from jax.experimental.pallas import tpu_sc as plsc

HID = 128
NH = 4
DH = HID // NH
LW = HID + 16

NC, NS = 2, 16
NW = NC * NS

BB = 400

A1, OFF_SB1, OFF_SI1, PAD1 = 12288, 10000, 11200, 11600
NB_S, NB_B, NB_I = 25, 3, 1
NB1 = NB_S + NB_B + NB_I
A2, OFF_IS2, PAD2 = 2048, 1200, 1600
A3, PAD3 = 1024, 400

_f32 = jnp.float32


def _round_up(x, m):
    return (x + m - 1) // m * m


_G = np.kron(np.eye(NH), np.ones((DH, DH))).astype(np.float32)
_P = np.kron(np.eye(NH), np.ones((DH, NH)) / DH).astype(np.float32)
_Q = np.kron(np.eye(NH), np.ones((NH, DH)) / NH).astype(np.float32)
_H = (np.tile(np.eye(DH), (NH, 1)) / NH).astype(np.float32)

_TBL_KEYS = ('wsT', 'bs', 'wmbig', 'bmrep', 'warep')
_RGA_KEYS = ('m1', 'bmerge', 'woaT', 'wobT', 'bout', 'g', 'b')


def _prep_rga(p):
    ws, bs = p['src_proj']
    wm, bm = p['msg_fc']
    wa = p['attn_fc'][0][0]
    wmerge, bmerge = p['merge_fc']
    wo, bo = p['out_fc']
    g, b = p['norm']
    eye = jnp.eye(NH, dtype=_f32)
    return dict(
        wsT=ws.T,
        bs=bs[None, :],
        wmbig=jnp.kron(eye, wm.T),
        bmrep=jnp.tile(bm, NH)[None, :],
        warep=jnp.tile(wa[:DH], NH)[None, :],
        m1=_H @ wmerge.T,
        bmerge=bmerge[None, :],
        woaT=wo[:, :HID].T,
        wobT=wo[:, HID:].T,
        bout=bo[None, :],
        g=g[None, :],
        b=b[None, :],
    )


def _stack(ws, keys):
    return tuple(jnp.stack([w[k] for w in ws]) for k in keys)


def _full(a):
    return pl.BlockSpec(a.shape, lambda i: (0,) * a.ndim)


def _wspec(a, relmap):
    return pl.BlockSpec((1,) + a.shape[1:], lambda i: (relmap(i),) + (0,) * (a.ndim - 1))


def _enc_body(x, w1T, b1, w2T, b2, o):
    h = jnp.maximum(x[...] @ w1T[0] + b1[0], 0.0)
    o[...] = h @ w2T[0] + b2[0]


def _enc_all(xfeat, params):
    def padw(p, f):
        (w1, b1), (w2, b2) = p
        w1T = jnp.pad(w1.T, ((0, 64 - w1.shape[1]), (0, 0)))
        return w1T, b1[None, :], w2.T, b2[None, :]

    stacked = [padw(params[k], 64) for k in ('stock_enc', 'bank_enc', 'industry_enc')]
    w1T, b1, w2T, b2 = (jnp.stack([s[j] for s in stacked]) for j in range(4))
    relmap = lambda i: jnp.where(i < NB_S, 0, jnp.where(i < NB_S + NB_B, 1, 2))
    return pl.pallas_call(
        _enc_body,
        grid=(NB1,),
        in_specs=[pl.BlockSpec((BB, 64), lambda i: (i, 0)),
                  _wspec(w1T, relmap), _wspec(b1, relmap),
                  _wspec(w2T, relmap), _wspec(b2, relmap)],
        out_specs=pl.BlockSpec((BB, HID), lambda i: (i, 0)),
        out_shape=jax.ShapeDtypeStruct((A1, HID), _f32),
    )(xfeat, w1T, b1, w2T, b2)


def _tbl_body(x, wsT, bs, wmbig, bmrep, warep, G, P, o):
    sf = x[...] @ wsT[0] + bs[0]
    t = jnp.tanh(sf)
    ab = (t * warep[0]) @ G[...]
    scale = jnp.exp(ab)
    wmsg = (sf @ wmbig[0] + bmrep[0]) * scale
    den16 = scale @ P[...]
    o[...] = jnp.concatenate([wmsg, den16], axis=1)


def _tbl(x, wlist, relmap, xmap, nblocks, out_rows):
    ws = _stack(wlist, _TBL_KEYS)
    return pl.pallas_call(
        _tbl_body,
        grid=(nblocks,),
        in_specs=[pl.BlockSpec((BB, HID), lambda i: (xmap(i), 0))]
        + [_wspec(a, relmap) for a in ws] + [_full(_G), _full(_P)],
        out_specs=pl.BlockSpec((BB, LW), lambda i: (i, 0)),
        out_shape=jax.ShapeDtypeStruct((out_rows, LW), _f32),
    )(x, *ws, _G, _P)


def _epi_math(acc, x, w, q):
    den_b = acc[:, HID:] @ q + 1e-12
    r = acc[:, :HID] / den_b
    merge = r @ w['m1'] + w['bmerge']
    upd = x @ w['woaT'] + merge @ w['wobT'] + w['bout']
    y = x + upd
    mu = jnp.mean(y, -1, keepdims=True)
    var = jnp.mean((y - mu) ** 2, -1, keepdims=True)
    return (y - mu) * lax.rsqrt(var + 1e-5) * w['g'] + w['b']


def _epi_body(x, a0, a1, q, m1, bmerge, woaT, wobT, bout, g, b, o):
    w = dict(m1=m1[0], bmerge=bmerge[0], woaT=woaT[0], wobT=wobT[0],
             bout=bout[0], g=g[0], b=b[0])
    o[...] = _epi_math(a0[0] + a1[0], x[...], w, q[...])


def _epi(state, acc, wlist, relmap, accmap, stmap, nblocks):
    ws = _stack(wlist, _RGA_KEYS)
    return pl.pallas_call(
        _epi_body,
        grid=(nblocks,),
        in_specs=[
            pl.BlockSpec((BB, HID), lambda i: (stmap(i), 0)),
            pl.BlockSpec((1, BB, LW), lambda i: (0, accmap(i), 0)),
            pl.BlockSpec((1, BB, LW), lambda i: (1, accmap(i), 0)),
            _full(_Q),
        ] + [_wspec(a, relmap) for a in ws],
        out_specs=pl.BlockSpec((BB, HID), lambda i: (stmap(i), 0)),
        out_shape=jax.ShapeDtypeStruct(state.shape, _f32),
        input_output_aliases={0: 0},
    )(state, acc, acc, _Q, *ws)


def _mega_body(x, ss0, ss1, sb0, sb1, si0, si1, q, *flat, nsb, nsi):
    wss = dict(zip(_RGA_KEYS, (f[...] for f in flat[0:7])))
    wsb = dict(zip(_RGA_KEYS, (f[...] for f in flat[7:14])))
    wsi = dict(zip(_RGA_KEYS, (f[...] for f in flat[14:21])))
    wpT, bp, wsrep, woT, bo, gm, bm = flat[21:28]
    o = flat[28]
    i = pl.program_id(0)
    msb = jnp.where(i < nsb, 1.0, 0.0).astype(_f32)
    msi = jnp.where(i < nsi, 1.0, 0.0).astype(_f32)
    xv = x[...]
    qv = q[...]
    p0 = xv
    p1 = _epi_math(ss0[0] + ss1[0], xv, wss, qv)
    p2 = _epi_math((sb0[0] + sb1[0]) * msb, xv, wsb, qv)
    p3 = _epi_math((si0[0] + si1[0]) * msi, xv, wsi, qv)
    paths = (p0, p1, p2, p3)
    scores = [jnp.sum(jnp.tanh(pp @ wpT[...] + bp[...]) * wsrep[...], -1, keepdims=True)
              for pp in paths]
    m = jnp.maximum(jnp.maximum(scores[0], scores[1]),
                    jnp.maximum(scores[2], scores[3]))
    es = [jnp.exp(s - m) for s in scores]
    den = es[0] + es[1] + es[2] + es[3]
    mix = sum(e * pp for e, pp in zip(es, paths)) / den
    y = mix @ woT[...] + bo[...]
    mu = jnp.mean(y, -1, keepdims=True)
    var = jnp.mean((y - mu) ** 2, -1, keepdims=True)
    o[...] = (y - mu) * lax.rsqrt(var + 1e-5) * gm[...] + bm[...]


def _mega(state, acc, wss, wsb, wsi, meta):
    sb0, si0 = OFF_SB1 // BB, OFF_SI1 // BB
    wargs = (tuple(wss[k] for k in _RGA_KEYS) + tuple(wsb[k] for k in _RGA_KEYS)
             + tuple(wsi[k] for k in _RGA_KEYS)
             + (meta['path_fc'][0].T, meta['path_fc'][1][None, :],
                meta['score_fc'][0][None, :], meta['out_fc'][0].T,
                meta['out_fc'][1][None, :], meta['norm'][0][None, :],
                meta['norm'][1][None, :]))
    return pl.pallas_call(
        functools.partial(_mega_body, nsb=NB_B, nsi=NB_I),
        grid=(NB_S,),
        in_specs=[
            pl.BlockSpec((BB, HID), lambda i: (i, 0)),
            pl.BlockSpec((1, BB, LW), lambda i: (0, i, 0)),
            pl.BlockSpec((1, BB, LW), lambda i: (1, i, 0)),
            pl.BlockSpec((1, BB, LW), lambda i: (0, sb0 + jnp.minimum(i, NB_B - 1), 0)),
            pl.BlockSpec((1, BB, LW), lambda i: (1, sb0 + jnp.minimum(i, NB_B - 1), 0)),
            pl.BlockSpec((1, BB, LW), lambda i: (0, si0 + jnp.minimum(i, NB_I - 1), 0)),
            pl.BlockSpec((1, BB, LW), lambda i: (1, si0 + jnp.minimum(i, NB_I - 1), 0)),
            _full(_Q),
        ] + [_full(a) for a in wargs],
        out_specs=pl.BlockSpec((BB, HID), lambda i: (i, 0)),
        out_shape=jax.ShapeDtypeStruct(state.shape, _f32),
        input_output_aliases={0: 0},
    )(state, acc, acc, acc, acc, acc, acc, _Q, *wargs)


def _win_for(A):
    return 32 if A > 4096 else 64


@functools.cache
def _make_scagg(A, E):
    win = _win_for(A)
    C = E // NW
    nwin = C // win
    assert C % win == 0 and nwin % 4 == 0
    niter = nwin // 4
    rz = A // NS
    assert rz % 8 == 0
    zc = next(c for c in (128, 64, 32, 16, 8) if rz % c == 0 and c <= win)
    mesh = plsc.VectorSubcoreMesh(core_axis_name="c", subcore_axis_name="s",
                                  num_cores=NC, num_subcores=NS)

    @functools.partial(
        pl.kernel,
        out_type=jax.ShapeDtypeStruct((NC, A, LW), _f32),
        mesh=mesh,
        compiler_params=pltpu.CompilerParams(use_tc_tiling_on_sc=False),
        scratch_types=[
            pltpu.VMEM((3, 8, win), jnp.int32),
            pltpu.VMEM((4, win, LW), _f32),
            pltpu.VMEM_SHARED((A, LW), _f32),
            [pltpu.SemaphoreType.DMA] * 4,
            [pltpu.SemaphoreType.DMA] * 4,
        ],
    )
    def scagg(table, idxs, zrows, out, idxb, rows, accum, gs, ss):
        c = lax.axis_index("c")
        s = lax.axis_index("s")
        wid = s * NC + c
        pltpu.sync_copy(zrows.at[pl.ds(0, zc)], rows.at[0, pl.ds(0, zc)])
        for j in range(rz // zc):
            pltpu.sync_copy(rows.at[0, pl.ds(0, zc)],
                            accum.at[pl.ds(s * rz + j * zc, zc)])
        plsc.subcore_barrier()
        gbase = wid * niter
        pltpu.sync_copy(idxs.at[gbase], idxb.at[0])
        pltpu.async_copy(table.at[idxb.at[0, 0]], rows.at[0], gs[0])
        pltpu.async_copy(table.at[idxb.at[0, 1]], rows.at[1], gs[1])

        def dwait(sem, buf):
            pltpu.make_async_copy(table.at[pl.ds(0, win)], rows.at[buf], sem).wait()

        def body(jj, carry):
            cur = jj % 3
            nxt = (jj + 1) % 3

            @pl.when(jj < niter - 1)
            def _():
                pltpu.sync_copy(idxs.at[gbase + jj + 1], idxb.at[nxt])

            for b in range(4):
                fb = (b + 2) % 4
                if b < 2:
                    @pl.when(jj > 0)
                    def _():
                        dwait(ss[fb], fb)
                        pltpu.async_copy(table.at[idxb.at[cur, b + 2]],
                                         rows.at[fb], gs[fb])

                    @pl.when(jj == 0)
                    def _():
                        pltpu.async_copy(table.at[idxb.at[cur, b + 2]],
                                         rows.at[fb], gs[fb])
                else:
                    dwait(ss[fb], fb)

                    @pl.when(jj < niter - 1)
                    def _():
                        pltpu.async_copy(table.at[idxb.at[nxt, b - 2]],
                                         rows.at[fb], gs[fb])
                dwait(gs[b], b)
                pltpu.async_copy(rows.at[b], accum.at[idxb.at[cur, 4 + b]],
                                 ss[b], add=True)
            return carry

        lax.fori_loop(0, niter, body, 0)
        dwait(ss[2], 2)
        dwait(ss[3], 3)
        plsc.subcore_barrier()
        pltpu.sync_copy(accum.at[pl.ds(s * rz, rz)], out.at[c, pl.ds(s * rz, rz)])

    return scagg


def _scagg(table, src, dst, A, E):
    win = _win_for(A)
    s4 = src.reshape(-1, 4, win)
    d4 = dst.reshape(-1, 4, win)
    idxs = jnp.concatenate([s4, d4], axis=1)
    zrows = jnp.zeros((128, LW), _f32)
    return _make_scagg(A, E)(table, idxs, zrows)


def _cat_edges(parts, pad_base, total):
    srcs = [e[0].astype(jnp.int32) + off for e, off in parts]
    dsts = [e[1].astype(jnp.int32) + off for e, off in parts]
    ne = sum(s.shape[0] for s in srcs)
    npad = total - ne
    pad = pad_base + (jnp.arange(npad, dtype=jnp.int32) % 8)
    return jnp.concatenate(srcs + [pad]), jnp.concatenate(dsts + [pad])


def kernel(stock_feat, bank_feat, industry_feat, edge_index_ss, edge_index_sb,
           edge_index_si, edge_index_bs, edge_index_is, edge_index_ii, params):
    xfeat = jnp.concatenate([
        stock_feat,
        jnp.pad(bank_feat, ((0, 200), (0, 32))),
        jnp.pad(industry_feat, ((0, 300), (0, 48))),
        jnp.zeros((A1 - PAD1, 64), _f32),
    ])
    state = _enc_all(xfeat, params)

    e1 = _round_up(edge_index_ss.shape[1] + edge_index_sb.shape[1]
                   + edge_index_si.shape[1], NW * 4 * _win_for(A1))
    src1, dst1 = _cat_edges([(edge_index_ss, 0), (edge_index_sb, OFF_SB1),
                             (edge_index_si, OFF_SI1)], PAD1, e1)
    e2 = _round_up(edge_index_bs.shape[1] + edge_index_is.shape[1],
                   NW * 4 * _win_for(A2))
    src2, dst2 = _cat_edges([(edge_index_bs, 0), (edge_index_is, OFF_IS2)], PAD2, e2)
    e3 = _round_up(edge_index_ii.shape[1], NW * 4 * _win_for(A3))
    src3, dst3 = _cat_edges([(edge_index_ii, 0)], PAD3, e3)

    rel1 = lambda i: jnp.where(i < NB_S, 0, jnp.where(i < NB_S + NB_B, 1, 2))
    rel2 = lambda i: jnp.where(i < 3, 0, 1)
    x2map = lambda i: jnp.where(i < 3, i, 0)

    layers = params['layers']
    for li, lp in enumerate(layers):
        wss, wsb, wsi = _prep_rga(lp['SS']), _prep_rga(lp['SB']), _prep_rga(lp['SI'])

        t1 = _tbl(state, [wss, wsb, wsi], rel1, lambda i: i, NB1, A1)
        acc1 = _scagg(t1, src1, dst1, A1, e1)
        state = _mega(state, acc1, wss, wsb, wsi, params['meta'])

        if li == len(layers) - 1:
            break
        wbs, wis, wii = _prep_rga(lp['BS']), _prep_rga(lp['IS']), _prep_rga(lp['II'])

        t2 = _tbl(state, [wbs, wis], rel2, x2map, 4, A2)
        acc2 = _scagg(t2, src2, dst2, A2, e2)
        state = _epi(state, acc2, [wbs, wis], rel2, lambda i: i,
                     lambda i: NB_S + i, 4)

        t3 = _tbl(state, [wii], lambda i: 0, lambda i: NB_S + NB_B, 1, A3)
        acc3 = _scagg(t3, src3, dst3, A3, e3)
        state = _epi(state, acc3, [wii], lambda i: 0, lambda i: 0,
                     lambda i: NB_S + NB_B, 1)

    return state[:OFF_SB1]
